# baseline (speedup 1.0000x reference)
import functools

import jax
import jax.numpy as jnp
from jax import lax
from jax.experimental import pallas as pl
from jax.experimental.pallas import tpu as pltpu
from jax.experimental.pallas import tpu_sc as plsc

N = 50000
E = 800000
D = 64
W = 128

NPAD = 51200
EPAD = 819200
NW = 32
EHALF = EPAD // 2
EW = EHALF // NW
GC = 256
GCH = EW // GC
ET = EHALF // 16

_mesh = plsc.VectorSubcoreMesh(core_axis_name="c", subcore_axis_name="s")
_bf16 = jnp.bfloat16


@functools.partial(
    pl.kernel,
    mesh=_mesh,
    compiler_params=pltpu.CompilerParams(use_tc_tiling_on_sc=False),
    out_type=jax.ShapeDtypeStruct((256,), jnp.int32),
    scratch_types=[
        pltpu.VMEM((1, 128), jnp.int32),
        pltpu.VMEM((128, 128), jnp.int32),
        pltpu.SemaphoreType.DMA,
    ],
)
def _warmup_sc(in_hbm, out_hbm, idxbuf, buf, sem):
    c = lax.axis_index("c")
    s = lax.axis_index("s")
    wid = s * 2 + c
    pltpu.sync_copy(in_hbm.at[pl.ds(pl.multiple_of(wid * 8, 8), 1)], idxbuf)

    def imask(j, _):
        idxbuf[0, pl.ds(j * 16, 16)] = idxbuf[0, pl.ds(j * 16, 16)] & 4095
        return 0

    lax.fori_loop(0, 8, imask, 0)

    def spin(g, _):
        pltpu.async_copy(in_hbm.at[idxbuf.at[0]], buf, sem).wait()
        return 0

    lax.fori_loop(0, 48, spin, 0)

    @pl.when(s == 0)
    def _():
        off = pl.multiple_of(c * 128, 8)
        pltpu.sync_copy(buf.at[0], out_hbm.at[pl.ds(off, 128)])


@functools.partial(
    pl.kernel,
    mesh=_mesh,
    compiler_params=pltpu.CompilerParams(use_tc_tiling_on_sc=False),
    out_type=jax.ShapeDtypeStruct((EHALF, W), jnp.float32),
    scratch_types=[
        pltpu.VMEM((GC,), jnp.int32),
        pltpu.VMEM((GC,), jnp.int32),
        pltpu.VMEM((GC, W), jnp.float32),
        pltpu.VMEM((GC, W), jnp.float32),
        pltpu.SemaphoreType.DMA,
    ],
)
def _gather_sc(ta0_hbm, tb0_hbm, ta1_hbm, tb1_hbm, row_hbm, col_hbm, wu_hbm,
               sr_out, rowv, colv, bufa, bufb, sem):
    del wu_hbm
    c = lax.axis_index("c")
    s = lax.axis_index("s")
    base = (s * 2 + c) * EW

    def run(ta_hbm, tb_hbm):
        def chunk(g, _):
            eb = pl.multiple_of(base + g * GC, 8)
            pltpu.sync_copy(row_hbm.at[pl.ds(eb, GC)], rowv)
            pltpu.sync_copy(col_hbm.at[pl.ds(eb, GC)], colv)
            h1 = pltpu.async_copy(ta_hbm.at[rowv], bufa, sem)
            h2 = pltpu.async_copy(tb_hbm.at[colv], bufb, sem)
            h1.wait()
            h2.wait()

            def vrow(i, _):
                r = i * 4
                for rr in range(4):
                    for j in range(4):
                        cs = pl.ds(j * 16, 16)
                        bufa[r + rr, cs] = bufa[r + rr, cs] + bufb[r + rr, cs]
                    cs = pl.ds(64, 16)
                    bufa[r + rr, cs] = bufa[r + rr, cs] - bufb[r + rr, cs]
                return 0

            lax.fori_loop(0, GC // 4, vrow, 0)
            pltpu.sync_copy(bufa, sr_out.at[pl.ds(eb, GC)])
            return 0

        lax.fori_loop(0, GCH, chunk, 0)

    @pl.when(c == 0)
    def _():
        run(ta0_hbm, tb0_hbm)

    @pl.when(c == 1)
    def _():
        run(ta1_hbm, tb1_hbm)


HALF = 25000
RSH = 25088
SSUB = 128
SCH = ET // SSUB


@functools.partial(
    pl.kernel,
    mesh=_mesh,
    compiler_params=pltpu.CompilerParams(use_tc_tiling_on_sc=False,
                                         needs_layout_passes=False),
    out_type=jax.ShapeDtypeStruct((N, W), _bf16),
    scratch_types=[
        pltpu.VMEM((SSUB,), jnp.int32),
        pltpu.VMEM((SSUB,), jnp.int32),
        pltpu.VMEM((SSUB, W), jnp.float32),
        pltpu.VMEM((SSUB, W), _bf16),
        pltpu.VMEM_SHARED((RSH, W), _bf16),
    ],
)
def _scatter_sc(mc_hbm, col_hbm, acc_out, colv, idxv, loadbuf, databuf, acc_sh):
    c = lax.axis_index("c")
    s = lax.axis_index("s")
    nbase = c * HALF

    def zrow(i, _):
        for j in range(W // 32):
            databuf[i, pl.ds(j * 32, 32)] = jnp.zeros((32,), _bf16)
        return 0

    lax.fori_loop(0, SSUB, zrow, 0)
    for q in range(12):
        r0 = pl.multiple_of(s * (RSH // 16) + q * SSUB, 8)
        pltpu.sync_copy(databuf, acc_sh.at[pl.ds(r0, SSUB)])
    r0 = pl.multiple_of(s * (RSH // 16) + 12 * SSUB, 8)
    pltpu.sync_copy(databuf.at[pl.ds(0, 32)], acc_sh.at[pl.ds(r0, 32)])
    plsc.subcore_barrier()

    def chunk(g, _):
        eb = pl.multiple_of(s * ET + g * SSUB, 8)
        pltpu.sync_copy(col_hbm.at[pl.ds(eb, SSUB)], colv)

        def ibody(i, _):
            v = colv[pl.ds(i * 16, 16)]
            l = v - nbase
            ok = (l >= 0) & (l < HALF)
            idxv[pl.ds(i * 16, 16)] = jnp.where(ok, l, HALF)
            return 0

        lax.fori_loop(0, SSUB // 16, ibody, 0)
        pltpu.sync_copy(mc_hbm.at[pl.ds(eb, SSUB)], loadbuf)

        def prow(i, _):
            for rr in range(2):
                r = i * 2 + rr
                for g in range(3):
                    a = loadbuf[r, pl.ds(g * 32, 16)]
                    b = loadbuf[r, pl.ds(g * 32 + 16, 16)]
                    databuf[r, pl.ds(g * 32, 32)] = plsc.pack(
                        a, b, format=plsc.PackFormat.INTERLEAVED)
            return 0

        lax.fori_loop(0, SSUB // 2, prow, 0)
        pltpu.sync_copy(databuf, acc_sh.at[idxv], add=True)
        return 0

    lax.fori_loop(0, SCH, chunk, 0)
    plsc.subcore_barrier()

    ob = s * 1560
    og = nbase + s * 1560
    pltpu.sync_copy(acc_sh.at[pl.ds(ob, 1560)], acc_out.at[pl.ds(og, 1560)])

    @pl.when(s == 0)
    def _():
        pltpu.sync_copy(acc_sh.at[pl.ds(24960, 40)], acc_out.at[pl.ds(nbase + 24960, 40)])


def _silu(v):
    return v * jax.nn.sigmoid(v)


def _pre_body(h_ref, x_ref, w1r_ref, w1c_ref, ta0_ref, tb0_ref, ta1_ref, tb1_ref):
    h = h_ref[...]
    x16 = x_ref[...]
    z = jnp.zeros((h.shape[0], W - D - 16), jnp.float32)
    ta = jnp.concatenate(
        [jnp.dot(h, w1r_ref[...], preferred_element_type=jnp.float32), x16, z],
        axis=1)
    tb = jnp.concatenate(
        [jnp.dot(h, w1c_ref[...], preferred_element_type=jnp.float32), x16, z],
        axis=1)
    ta0_ref[...] = ta
    tb0_ref[...] = tb
    ta1_ref[...] = ta
    tb1_ref[...] = tb


def _edge_body(sr_ref, w1d_ref, be1_ref, we2_ref, be2_ref,
               wc1_ref, bc1_ref, wc2_ref, mc_ref):
    sr = sr_ref[...]
    rel = sr[:, D:D + 16]
    dist = jnp.sqrt(jnp.sum(rel * rel, axis=-1, keepdims=True))
    pre = sr[:, :D] + dist * w1d_ref[...] + be1_ref[...]
    t = _silu(pre)
    msg = jnp.dot(t, we2_ref[...], preferred_element_type=jnp.float32) + be2_ref[...]
    cpre = jnp.dot(msg, wc1_ref[...], preferred_element_type=jnp.float32) + bc1_ref[...]
    cw = jnp.sum(_silu(cpre) * wc2_ref[...], axis=-1, keepdims=True)
    z = jnp.zeros((msg.shape[0], W - D - 16), jnp.float32)
    mc_ref[...] = jnp.concatenate([msg, rel * cw, z], axis=1)


_PSRC = []
for _g in range(4):
    for _i in range(16):
        _PSRC.extend([32 * _g + _i, 32 * _g + 16 + _i])
_XIDX = tuple(_PSRC.index(D + k) for k in range(16))


def _node_body(h_ref, acc1_ref, acc2_ref, x_ref, wn1a_ref, wn1bp_ref, bn1_ref,
               wn2_ref, bn2_ref, hn_ref, xn_ref):
    h = h_ref[...]
    acc = acc1_ref[...].astype(jnp.float32) + acc2_ref[...].astype(jnp.float32)
    pre = (jnp.dot(h, wn1a_ref[...], preferred_element_type=jnp.float32)
           + jnp.dot(acc[:, :D], wn1bp_ref[...], preferred_element_type=jnp.float32)
           + bn1_ref[...])
    hn_ref[...] = h + jnp.dot(_silu(pre), wn2_ref[...],
                              preferred_element_type=jnp.float32) + bn2_ref[...]
    xacc = jnp.concatenate([acc[:, j:j + 1] for j in _XIDX], axis=1)
    xn_ref[...] = x_ref[...] + xacc


def _full(shape):
    return pl.BlockSpec(shape, lambda i: (0, 0))


def kernel(h, x, edge_index, We1, be1, We2, be2, Wn1, bn1, Wn2, bn2, Wc1, bc1, Wc2):
    f32 = jnp.float32
    row = edge_index[0]
    col = edge_index[1]
    pad = jnp.full((EPAD - E,), N, dtype=jnp.int32)
    row_p = jnp.concatenate([row, pad])
    col_p = jnp.concatenate([col, pad])

    h_pad = jnp.zeros((NPAD, D), f32).at[:N].set(h)
    x_pad = jnp.zeros((NPAD, 16), f32).at[:N, :3].set(x)
    x16 = x_pad[:N]

    bn = 2048
    ta0, tb0, ta1, tb1 = pl.pallas_call(
        _pre_body,
        grid=(NPAD // bn,),
        in_specs=[pl.BlockSpec((bn, D), lambda i: (i, 0)),
                  pl.BlockSpec((bn, 16), lambda i: (i, 0)),
                  _full((D, D)), _full((D, D))],
        out_specs=[pl.BlockSpec((bn, W), lambda i: (i, 0))] * 4,
        out_shape=[jax.ShapeDtypeStruct((NPAD, W), f32)] * 4,
    )(h_pad, x_pad, We1[:D], We1[D:2 * D])

    be = 2048
    w1d = We1[2 * D:2 * D + 1]

    def edge_mlp(sr):
        return pl.pallas_call(
            _edge_body,
            grid=(EHALF // be,),
            in_specs=[pl.BlockSpec((be, W), lambda i: (i, 0)),
                      _full((1, D)), _full((1, D)), _full((D, D)), _full((1, D)),
                      _full((D, D)), _full((1, D)), _full((1, D))],
            out_specs=pl.BlockSpec((be, W), lambda i: (i, 0)),
            out_shape=jax.ShapeDtypeStruct((EHALF, W), jnp.float32),
        )(sr, w1d, be1.reshape(1, D), We2, be2.reshape(1, D),
          Wc1, bc1.reshape(1, D), Wc2.reshape(1, D))

    row1, row2 = row_p[:EHALF], row_p[EHALF:]
    col1, col2 = col_p[:EHALF], col_p[EHALF:]
    wu = _warmup_sc(col_p.reshape(EPAD // 128, 128))
    sr1 = _gather_sc(ta0, tb0, ta1, tb1, row1, col1, wu)
    sr2 = _gather_sc(ta0, tb0, ta1, tb1, row2, col2, wu)
    mc1 = edge_mlp(sr1)
    mc2 = edge_mlp(sr2)
    acc1 = _scatter_sc(mc1, col1)
    acc2 = _scatter_sc(mc2, col2)

    bnn = 2000
    _node_call = pl.pallas_call(
        _node_body,
        grid=(N // bnn,),
        in_specs=[pl.BlockSpec((bnn, D), lambda i: (i, 0)),
                  pl.BlockSpec((bnn, W), lambda i: (i, 0)),
                  pl.BlockSpec((bnn, W), lambda i: (i, 0)),
                  pl.BlockSpec((bnn, 16), lambda i: (i, 0)),
                  _full((D, D)), _full((D, D)), _full((1, D)),
                  _full((D, D)), _full((1, D))],
        out_specs=[pl.BlockSpec((bnn, D), lambda i: (i, 0)),
                   pl.BlockSpec((bnn, 16), lambda i: (i, 0))],
        out_shape=[jax.ShapeDtypeStruct((N, D), f32),
                   jax.ShapeDtypeStruct((N, 16), f32)],
    )
    wn1b_perm = Wn1[D:][jnp.array(_PSRC[:D]), :]
    h_new, x_new16 = _node_call(h, acc1, acc2, x16, Wn1[:D], wn1b_perm,
                                bn1.reshape(1, D), Wn2, bn2.reshape(1, D))

    return h_new, x_new16[:, :3]


# speedup vs baseline: 1.0087x; 1.0087x over previous
import functools

import jax
import jax.numpy as jnp
from jax import lax
from jax.experimental import pallas as pl
from jax.experimental.pallas import tpu as pltpu
from jax.experimental.pallas import tpu_sc as plsc

N = 50000
E = 800000
D = 64
W = 128

NPAD = 51200
EPAD = 819200
NW = 32
EHALF = EPAD // 2
EW = EHALF // NW
GC = 256
GCH = EW // GC
ET = EHALF // 16

_mesh = plsc.VectorSubcoreMesh(core_axis_name="c", subcore_axis_name="s")
_bf16 = jnp.bfloat16


@functools.partial(
    pl.kernel,
    mesh=_mesh,
    compiler_params=pltpu.CompilerParams(use_tc_tiling_on_sc=False),
    out_type=jax.ShapeDtypeStruct((256,), jnp.int32),
    scratch_types=[pltpu.VMEM((1, 128), jnp.int32)],
)
def _warmup_sc(in_hbm, out_hbm, buf):
    c = lax.axis_index("c")
    s = lax.axis_index("s")

    @pl.when(s == 0)
    def _():
        pltpu.sync_copy(in_hbm.at[pl.ds(pl.multiple_of(c, 1), 1)], buf)
        pltpu.sync_copy(buf.at[0], out_hbm.at[pl.ds(pl.multiple_of(c * 128, 8), 128)])


@functools.partial(
    pl.kernel,
    mesh=_mesh,
    compiler_params=pltpu.CompilerParams(use_tc_tiling_on_sc=False),
    out_type=jax.ShapeDtypeStruct((EHALF, W), jnp.float32),
    scratch_types=[
        pltpu.VMEM((GC,), jnp.int32),
        pltpu.VMEM((GC,), jnp.int32),
        pltpu.VMEM((GC, W), jnp.float32),
        pltpu.VMEM((GC, W), jnp.float32),
        pltpu.SemaphoreType.DMA,
    ],
)
def _gather_sc(ta0_hbm, tb0_hbm, ta1_hbm, tb1_hbm, row_hbm, col_hbm, wu_hbm,
               sr_out, rowv, colv, bufa, bufb, sem):
    del wu_hbm
    c = lax.axis_index("c")
    s = lax.axis_index("s")
    base = (s * 2 + c) * EW

    def run(ta_hbm, tb_hbm):
        def chunk(g, _):
            eb = pl.multiple_of(base + g * GC, 8)
            pltpu.sync_copy(row_hbm.at[pl.ds(eb, GC)], rowv)
            pltpu.sync_copy(col_hbm.at[pl.ds(eb, GC)], colv)
            h1 = pltpu.async_copy(ta_hbm.at[rowv], bufa, sem)
            h2 = pltpu.async_copy(tb_hbm.at[colv], bufb, sem)
            h1.wait()
            h2.wait()

            def vrow(i, _):
                r = i * 4
                for rr in range(4):
                    for j in range(4):
                        cs = pl.ds(j * 16, 16)
                        bufa[r + rr, cs] = bufa[r + rr, cs] + bufb[r + rr, cs]
                    cs = pl.ds(64, 16)
                    bufa[r + rr, cs] = bufa[r + rr, cs] - bufb[r + rr, cs]
                return 0

            lax.fori_loop(0, GC // 4, vrow, 0)
            pltpu.sync_copy(bufa, sr_out.at[pl.ds(eb, GC)])
            return 0

        lax.fori_loop(0, GCH, chunk, 0)

    @pl.when(c == 0)
    def _():
        run(ta0_hbm, tb0_hbm)

    @pl.when(c == 1)
    def _():
        run(ta1_hbm, tb1_hbm)


HALF = 25000
RSH = 25088
SSUB = 128
SCH = ET // SSUB


@functools.partial(
    pl.kernel,
    mesh=_mesh,
    compiler_params=pltpu.CompilerParams(use_tc_tiling_on_sc=False,
                                         needs_layout_passes=False),
    out_type=jax.ShapeDtypeStruct((N, W), _bf16),
    scratch_types=[
        pltpu.VMEM((SSUB,), jnp.int32),
        pltpu.VMEM((SSUB,), jnp.int32),
        pltpu.VMEM((SSUB, W), jnp.float32),
        pltpu.VMEM((SSUB, W), _bf16),
        pltpu.VMEM_SHARED((RSH, W), _bf16),
    ],
)
def _scatter_sc(mc_hbm, col_hbm, acc_out, colv, idxv, loadbuf, databuf, acc_sh):
    c = lax.axis_index("c")
    s = lax.axis_index("s")
    nbase = c * HALF

    def zrow(i, _):
        for j in range(W // 32):
            databuf[i, pl.ds(j * 32, 32)] = jnp.zeros((32,), _bf16)
        return 0

    lax.fori_loop(0, SSUB, zrow, 0)
    for q in range(12):
        r0 = pl.multiple_of(s * (RSH // 16) + q * SSUB, 8)
        pltpu.sync_copy(databuf, acc_sh.at[pl.ds(r0, SSUB)])
    r0 = pl.multiple_of(s * (RSH // 16) + 12 * SSUB, 8)
    pltpu.sync_copy(databuf.at[pl.ds(0, 32)], acc_sh.at[pl.ds(r0, 32)])
    plsc.subcore_barrier()

    def chunk(g, _):
        eb = pl.multiple_of(s * ET + g * SSUB, 8)
        pltpu.sync_copy(col_hbm.at[pl.ds(eb, SSUB)], colv)

        def ibody(i, _):
            v = colv[pl.ds(i * 16, 16)]
            l = v - nbase
            ok = (l >= 0) & (l < HALF)
            idxv[pl.ds(i * 16, 16)] = jnp.where(ok, l, HALF)
            return 0

        lax.fori_loop(0, SSUB // 16, ibody, 0)
        pltpu.sync_copy(mc_hbm.at[pl.ds(eb, SSUB)], loadbuf)

        def prow(i, _):
            for rr in range(2):
                r = i * 2 + rr
                for g in range(3):
                    a = loadbuf[r, pl.ds(g * 32, 16)]
                    b = loadbuf[r, pl.ds(g * 32 + 16, 16)]
                    databuf[r, pl.ds(g * 32, 32)] = plsc.pack(
                        a, b, format=plsc.PackFormat.INTERLEAVED)
            return 0

        lax.fori_loop(0, SSUB // 2, prow, 0)
        pltpu.sync_copy(databuf, acc_sh.at[idxv], add=True)
        return 0

    lax.fori_loop(0, SCH, chunk, 0)
    plsc.subcore_barrier()

    ob = s * 1560
    og = nbase + s * 1560
    pltpu.sync_copy(acc_sh.at[pl.ds(ob, 1560)], acc_out.at[pl.ds(og, 1560)])

    @pl.when(s == 0)
    def _():
        pltpu.sync_copy(acc_sh.at[pl.ds(24960, 40)], acc_out.at[pl.ds(nbase + 24960, 40)])


def _silu(v):
    return v * jax.nn.sigmoid(v)


def _pre_body(h_ref, x_ref, w1r_ref, w1c_ref, ta0_ref, tb0_ref, ta1_ref, tb1_ref):
    h = h_ref[...]
    x16 = x_ref[...]
    z = jnp.zeros((h.shape[0], W - D - 16), jnp.float32)
    ta = jnp.concatenate(
        [jnp.dot(h, w1r_ref[...], preferred_element_type=jnp.float32), x16, z],
        axis=1)
    tb = jnp.concatenate(
        [jnp.dot(h, w1c_ref[...], preferred_element_type=jnp.float32), x16, z],
        axis=1)
    ta0_ref[...] = ta
    tb0_ref[...] = tb
    ta1_ref[...] = ta
    tb1_ref[...] = tb


def _edge_body(sr_ref, w1d_ref, be1_ref, we2_ref, be2_ref,
               wc1_ref, bc1_ref, wc2_ref, mc_ref):
    sr = sr_ref[...]
    rel = sr[:, D:D + 16]
    dist = jnp.sqrt(jnp.sum(rel * rel, axis=-1, keepdims=True))
    pre = sr[:, :D] + dist * w1d_ref[...] + be1_ref[...]
    t = _silu(pre)
    msg = jnp.dot(t, we2_ref[...], preferred_element_type=jnp.float32) + be2_ref[...]
    cpre = jnp.dot(msg, wc1_ref[...], preferred_element_type=jnp.float32) + bc1_ref[...]
    cw = jnp.sum(_silu(cpre) * wc2_ref[...], axis=-1, keepdims=True)
    z = jnp.zeros((msg.shape[0], W - D - 16), jnp.float32)
    mc_ref[...] = jnp.concatenate([msg, rel * cw, z], axis=1)


_PSRC = []
for _g in range(4):
    for _i in range(16):
        _PSRC.extend([32 * _g + _i, 32 * _g + 16 + _i])
_XIDX = tuple(_PSRC.index(D + k) for k in range(16))


def _node_body(h_ref, acc1_ref, acc2_ref, x_ref, wn1a_ref, wn1bp_ref, bn1_ref,
               wn2_ref, bn2_ref, hn_ref, xn_ref):
    h = h_ref[...]
    acc = acc1_ref[...].astype(jnp.float32) + acc2_ref[...].astype(jnp.float32)
    pre = (jnp.dot(h, wn1a_ref[...], preferred_element_type=jnp.float32)
           + jnp.dot(acc[:, :D], wn1bp_ref[...], preferred_element_type=jnp.float32)
           + bn1_ref[...])
    hn_ref[...] = h + jnp.dot(_silu(pre), wn2_ref[...],
                              preferred_element_type=jnp.float32) + bn2_ref[...]
    xacc = jnp.concatenate([acc[:, j:j + 1] for j in _XIDX], axis=1)
    xn_ref[...] = x_ref[...] + xacc


def _full(shape):
    return pl.BlockSpec(shape, lambda i: (0, 0))


def kernel(h, x, edge_index, We1, be1, We2, be2, Wn1, bn1, Wn2, bn2, Wc1, bc1, Wc2):
    f32 = jnp.float32
    row = edge_index[0]
    col = edge_index[1]
    pad = jnp.full((EPAD - E,), N, dtype=jnp.int32)
    row_p = jnp.concatenate([row, pad])
    col_p = jnp.concatenate([col, pad])

    h_pad = jnp.zeros((NPAD, D), f32).at[:N].set(h)
    x_pad = jnp.zeros((NPAD, 16), f32).at[:N, :3].set(x)
    x16 = x_pad[:N]

    bn = 2048
    ta0, tb0, ta1, tb1 = pl.pallas_call(
        _pre_body,
        grid=(NPAD // bn,),
        in_specs=[pl.BlockSpec((bn, D), lambda i: (i, 0)),
                  pl.BlockSpec((bn, 16), lambda i: (i, 0)),
                  _full((D, D)), _full((D, D))],
        out_specs=[pl.BlockSpec((bn, W), lambda i: (i, 0))] * 4,
        out_shape=[jax.ShapeDtypeStruct((NPAD, W), f32)] * 4,
    )(h_pad, x_pad, We1[:D], We1[D:2 * D])

    be = 2048
    w1d = We1[2 * D:2 * D + 1]

    def edge_mlp(sr):
        return pl.pallas_call(
            _edge_body,
            grid=(EHALF // be,),
            in_specs=[pl.BlockSpec((be, W), lambda i: (i, 0)),
                      _full((1, D)), _full((1, D)), _full((D, D)), _full((1, D)),
                      _full((D, D)), _full((1, D)), _full((1, D))],
            out_specs=pl.BlockSpec((be, W), lambda i: (i, 0)),
            out_shape=jax.ShapeDtypeStruct((EHALF, W), jnp.float32),
        )(sr, w1d, be1.reshape(1, D), We2, be2.reshape(1, D),
          Wc1, bc1.reshape(1, D), Wc2.reshape(1, D))

    row1, row2 = row_p[:EHALF], row_p[EHALF:]
    col1, col2 = col_p[:EHALF], col_p[EHALF:]
    wu = _warmup_sc(col_p.reshape(EPAD // 128, 128))
    sr1 = _gather_sc(ta0, tb0, ta1, tb1, row1, col1, wu)
    sr2 = _gather_sc(ta0, tb0, ta1, tb1, row2, col2, wu)
    mc1 = edge_mlp(sr1)
    mc2 = edge_mlp(sr2)
    acc1 = _scatter_sc(mc1, col1)
    acc2 = _scatter_sc(mc2, col2)

    bnn = 2000
    _node_call = pl.pallas_call(
        _node_body,
        grid=(N // bnn,),
        in_specs=[pl.BlockSpec((bnn, D), lambda i: (i, 0)),
                  pl.BlockSpec((bnn, W), lambda i: (i, 0)),
                  pl.BlockSpec((bnn, W), lambda i: (i, 0)),
                  pl.BlockSpec((bnn, 16), lambda i: (i, 0)),
                  _full((D, D)), _full((D, D)), _full((1, D)),
                  _full((D, D)), _full((1, D))],
        out_specs=[pl.BlockSpec((bnn, D), lambda i: (i, 0)),
                   pl.BlockSpec((bnn, 16), lambda i: (i, 0))],
        out_shape=[jax.ShapeDtypeStruct((N, D), f32),
                   jax.ShapeDtypeStruct((N, 16), f32)],
    )
    wn1b_perm = Wn1[D:][jnp.array(_PSRC[:D]), :]
    h_new, x_new16 = _node_call(h, acc1, acc2, x16, Wn1[:D], wn1b_perm,
                                bn1.reshape(1, D), Wn2, bn2.reshape(1, D))

    return h_new, x_new16[:, :3]


# speedup vs baseline: 1.0938x; 1.0843x over previous
import functools

import jax
import jax.numpy as jnp
from jax import lax
from jax.experimental import pallas as pl
from jax.experimental.pallas import tpu as pltpu
from jax.experimental.pallas import tpu_sc as plsc

N = 50000
E = 800000
D = 64
W = 128

NPAD = 51200
EPAD = 819200
NW = 32
EHALF = EPAD // 2
EW = EHALF // NW
GC = 256
GCH = EW // GC
ET = EHALF // 16

_mesh = plsc.VectorSubcoreMesh(core_axis_name="c", subcore_axis_name="s")
_bf16 = jnp.bfloat16


@functools.partial(
    pl.kernel,
    mesh=_mesh,
    compiler_params=pltpu.CompilerParams(use_tc_tiling_on_sc=False),
    out_type=jax.ShapeDtypeStruct((256,), jnp.int32),
    scratch_types=[pltpu.VMEM((128,), jnp.int32)],
)
def _warmup_sc(in_hbm, out_hbm, buf):
    c = lax.axis_index("c")
    s = lax.axis_index("s")

    @pl.when(s == 0)
    def _():
        off = pl.multiple_of(c * 128, 8)
        pltpu.sync_copy(in_hbm.at[pl.ds(off, 128)], buf)
        pltpu.sync_copy(buf, out_hbm.at[pl.ds(off, 128)])


@functools.partial(
    pl.kernel,
    mesh=_mesh,
    compiler_params=pltpu.CompilerParams(use_tc_tiling_on_sc=False),
    out_type=jax.ShapeDtypeStruct((EHALF, W), jnp.float32),
    scratch_types=[
        pltpu.VMEM((GC,), jnp.int32),
        pltpu.VMEM((GC,), jnp.int32),
        pltpu.VMEM((GC, W), jnp.float32),
        pltpu.VMEM((GC, W), jnp.float32),
        pltpu.SemaphoreType.DMA,
    ],
)
def _gather_sc(ta0_hbm, tb0_hbm, ta1_hbm, tb1_hbm, row_hbm, col_hbm, wu_hbm,
               sr_out, rowv, colv, bufa, bufb, sem):
    del wu_hbm
    c = lax.axis_index("c")
    s = lax.axis_index("s")
    base = (s * 2 + c) * EW

    def run(ta_hbm, tb_hbm):
        def chunk(g, _):
            eb = pl.multiple_of(base + g * GC, 8)
            pltpu.sync_copy(row_hbm.at[pl.ds(eb, GC)], rowv)
            pltpu.sync_copy(col_hbm.at[pl.ds(eb, GC)], colv)
            h1 = pltpu.async_copy(ta_hbm.at[rowv], bufa, sem)
            h2 = pltpu.async_copy(tb_hbm.at[colv], bufb, sem)
            h1.wait()
            h2.wait()

            def vrow(i, _):
                r = i * 4
                for rr in range(4):
                    for j in range(4):
                        cs = pl.ds(j * 16, 16)
                        bufa[r + rr, cs] = bufa[r + rr, cs] + bufb[r + rr, cs]
                    cs = pl.ds(64, 16)
                    bufa[r + rr, cs] = bufa[r + rr, cs] - bufb[r + rr, cs]
                return 0

            lax.fori_loop(0, GC // 4, vrow, 0)
            pltpu.sync_copy(bufa, sr_out.at[pl.ds(eb, GC)])
            return 0

        lax.fori_loop(0, GCH, chunk, 0)

    @pl.when(c == 0)
    def _():
        run(ta0_hbm, tb0_hbm)

    @pl.when(c == 1)
    def _():
        run(ta1_hbm, tb1_hbm)


HALF = 25000
RSH = 25088
SSUB = 128
SCH = ET // SSUB


@functools.partial(
    pl.kernel,
    mesh=_mesh,
    compiler_params=pltpu.CompilerParams(use_tc_tiling_on_sc=False,
                                         needs_layout_passes=False),
    out_type=jax.ShapeDtypeStruct((N, W), _bf16),
    scratch_types=[
        pltpu.VMEM((SSUB,), jnp.int32),
        pltpu.VMEM((SSUB,), jnp.int32),
        pltpu.VMEM((SSUB, W), jnp.float32),
        pltpu.VMEM((SSUB, W), _bf16),
        pltpu.VMEM_SHARED((RSH, W), _bf16),
    ],
)
def _scatter_sc(mc_hbm, col_hbm, acc_out, colv, idxv, loadbuf, databuf, acc_sh):
    c = lax.axis_index("c")
    s = lax.axis_index("s")
    nbase = c * HALF

    def zrow(i, _):
        for j in range(W // 32):
            databuf[i, pl.ds(j * 32, 32)] = jnp.zeros((32,), _bf16)
        return 0

    lax.fori_loop(0, SSUB, zrow, 0)
    for q in range(12):
        r0 = pl.multiple_of(s * (RSH // 16) + q * SSUB, 8)
        pltpu.sync_copy(databuf, acc_sh.at[pl.ds(r0, SSUB)])
    r0 = pl.multiple_of(s * (RSH // 16) + 12 * SSUB, 8)
    pltpu.sync_copy(databuf.at[pl.ds(0, 32)], acc_sh.at[pl.ds(r0, 32)])
    plsc.subcore_barrier()

    def chunk(g, _):
        eb = pl.multiple_of(s * ET + g * SSUB, 8)
        pltpu.sync_copy(col_hbm.at[pl.ds(eb, SSUB)], colv)

        def ibody(i, _):
            v = colv[pl.ds(i * 16, 16)]
            l = v - nbase
            ok = (l >= 0) & (l < HALF)
            idxv[pl.ds(i * 16, 16)] = jnp.where(ok, l, HALF)
            return 0

        lax.fori_loop(0, SSUB // 16, ibody, 0)
        pltpu.sync_copy(mc_hbm.at[pl.ds(eb, SSUB)], loadbuf)

        def prow(r, _):
            for g in range(3):
                a = loadbuf[r, pl.ds(g * 32, 16)]
                b = loadbuf[r, pl.ds(g * 32 + 16, 16)]
                databuf[r, pl.ds(g * 32, 32)] = plsc.pack(
                    a, b, format=plsc.PackFormat.INTERLEAVED)
            return 0

        lax.fori_loop(0, SSUB, prow, 0)
        pltpu.sync_copy(databuf, acc_sh.at[idxv], add=True)
        return 0

    lax.fori_loop(0, SCH, chunk, 0)
    plsc.subcore_barrier()

    ob = s * 1560
    og = nbase + s * 1560
    pltpu.sync_copy(acc_sh.at[pl.ds(ob, 1560)], acc_out.at[pl.ds(og, 1560)])

    @pl.when(s == 0)
    def _():
        pltpu.sync_copy(acc_sh.at[pl.ds(24960, 40)], acc_out.at[pl.ds(nbase + 24960, 40)])


def _silu(v):
    return v * jax.nn.sigmoid(v)


def _pre_body(h_ref, x_ref, w1r_ref, w1c_ref, ta0_ref, tb0_ref, ta1_ref, tb1_ref):
    h = h_ref[...]
    x16 = x_ref[...]
    z = jnp.zeros((h.shape[0], W - D - 16), jnp.float32)
    ta = jnp.concatenate(
        [jnp.dot(h, w1r_ref[...], preferred_element_type=jnp.float32), x16, z],
        axis=1)
    tb = jnp.concatenate(
        [jnp.dot(h, w1c_ref[...], preferred_element_type=jnp.float32), x16, z],
        axis=1)
    ta0_ref[...] = ta
    tb0_ref[...] = tb
    ta1_ref[...] = ta
    tb1_ref[...] = tb


def _edge_body(sr_ref, w1d_ref, be1_ref, we2_ref, be2_ref,
               wc1_ref, bc1_ref, wc2_ref, mc_ref):
    sr = sr_ref[...]
    rel = sr[:, D:D + 16]
    dist = jnp.sqrt(jnp.sum(rel * rel, axis=-1, keepdims=True))
    pre = sr[:, :D] + dist * w1d_ref[...] + be1_ref[...]
    t = _silu(pre)
    msg = jnp.dot(t, we2_ref[...], preferred_element_type=jnp.float32) + be2_ref[...]
    cpre = jnp.dot(msg, wc1_ref[...], preferred_element_type=jnp.float32) + bc1_ref[...]
    cw = jnp.sum(_silu(cpre) * wc2_ref[...], axis=-1, keepdims=True)
    z = jnp.zeros((msg.shape[0], W - D - 16), jnp.float32)
    mc_ref[...] = jnp.concatenate([msg, rel * cw, z], axis=1)


_PSRC = []
for _g in range(4):
    for _i in range(16):
        _PSRC.extend([32 * _g + _i, 32 * _g + 16 + _i])
_XIDX = tuple(_PSRC.index(D + k) for k in range(16))


def _node_body(h_ref, acc1_ref, acc2_ref, x_ref, wn1a_ref, wn1bp_ref, bn1_ref,
               wn2_ref, bn2_ref, hn_ref, xn_ref):
    h = h_ref[...]
    acc = acc1_ref[...].astype(jnp.float32) + acc2_ref[...].astype(jnp.float32)
    pre = (jnp.dot(h, wn1a_ref[...], preferred_element_type=jnp.float32)
           + jnp.dot(acc[:, :D], wn1bp_ref[...], preferred_element_type=jnp.float32)
           + bn1_ref[...])
    hn_ref[...] = h + jnp.dot(_silu(pre), wn2_ref[...],
                              preferred_element_type=jnp.float32) + bn2_ref[...]
    xacc = jnp.concatenate([acc[:, j:j + 1] for j in _XIDX], axis=1)
    xn_ref[...] = x_ref[...] + xacc


def _full(shape):
    return pl.BlockSpec(shape, lambda i: (0, 0))


def kernel(h, x, edge_index, We1, be1, We2, be2, Wn1, bn1, Wn2, bn2, Wc1, bc1, Wc2):
    f32 = jnp.float32
    row = edge_index[0]
    col = edge_index[1]
    pad = jnp.full((EPAD - E,), N, dtype=jnp.int32)
    row_p = jnp.concatenate([row, pad])
    col_p = jnp.concatenate([col, pad])

    h_pad = jnp.zeros((NPAD, D), f32).at[:N].set(h)
    x_pad = jnp.zeros((NPAD, 16), f32).at[:N, :3].set(x)
    x16 = x_pad[:N]

    bn = 2048
    ta0, tb0, ta1, tb1 = pl.pallas_call(
        _pre_body,
        grid=(NPAD // bn,),
        in_specs=[pl.BlockSpec((bn, D), lambda i: (i, 0)),
                  pl.BlockSpec((bn, 16), lambda i: (i, 0)),
                  _full((D, D)), _full((D, D))],
        out_specs=[pl.BlockSpec((bn, W), lambda i: (i, 0))] * 4,
        out_shape=[jax.ShapeDtypeStruct((NPAD, W), f32)] * 4,
    )(h_pad, x_pad, We1[:D], We1[D:2 * D])

    be = 2048
    w1d = We1[2 * D:2 * D + 1]

    def edge_mlp(sr):
        return pl.pallas_call(
            _edge_body,
            grid=(EHALF // be,),
            in_specs=[pl.BlockSpec((be, W), lambda i: (i, 0)),
                      _full((1, D)), _full((1, D)), _full((D, D)), _full((1, D)),
                      _full((D, D)), _full((1, D)), _full((1, D))],
            out_specs=pl.BlockSpec((be, W), lambda i: (i, 0)),
            out_shape=jax.ShapeDtypeStruct((EHALF, W), jnp.float32),
        )(sr, w1d, be1.reshape(1, D), We2, be2.reshape(1, D),
          Wc1, bc1.reshape(1, D), Wc2.reshape(1, D))

    row1, row2 = row_p[:EHALF], row_p[EHALF:]
    col1, col2 = col_p[:EHALF], col_p[EHALF:]
    wu = _warmup_sc(col_p[:256])
    sr1 = _gather_sc(ta0, tb0, ta1, tb1, row1, col1, wu)
    sr2 = _gather_sc(ta0, tb0, ta1, tb1, row2, col2, wu)
    mc1 = edge_mlp(sr1)
    mc2 = edge_mlp(sr2)
    acc1 = _scatter_sc(mc1, col1)
    acc2 = _scatter_sc(mc2, col2)

    bnn = 2000
    _node_call = pl.pallas_call(
        _node_body,
        grid=(N // bnn,),
        in_specs=[pl.BlockSpec((bnn, D), lambda i: (i, 0)),
                  pl.BlockSpec((bnn, W), lambda i: (i, 0)),
                  pl.BlockSpec((bnn, W), lambda i: (i, 0)),
                  pl.BlockSpec((bnn, 16), lambda i: (i, 0)),
                  _full((D, D)), _full((D, D)), _full((1, D)),
                  _full((D, D)), _full((1, D))],
        out_specs=[pl.BlockSpec((bnn, D), lambda i: (i, 0)),
                   pl.BlockSpec((bnn, 16), lambda i: (i, 0))],
        out_shape=[jax.ShapeDtypeStruct((N, D), f32),
                   jax.ShapeDtypeStruct((N, 16), f32)],
    )
    wn1b_perm = Wn1[D:][jnp.array(_PSRC[:D]), :]
    h_new, x_new16 = _node_call(h, acc1, acc2, x16, Wn1[:D], wn1b_perm,
                                bn1.reshape(1, D), Wn2, bn2.reshape(1, D))

    return h_new, x_new16[:, :3]
